# baseline (speedup 1.0000x reference)
import functools

import jax
import jax.numpy as jnp
from jax import lax
from jax.experimental import pallas as pl
from jax.experimental.pallas import tpu as pltpu
from jax.experimental.pallas import tpu_sc as plsc

N = 10000
E = 160000
EMB = 256
HALF = 128
FEAT = 512
NUM_LAYER = 5
NUM_GRAPHS = 256
EPS = 1e-5

NSUB = 16
NCORE = 2
CH = 128
NPAD = 10240
NCHUNK_N = 5
ECHUNKS = 84
EPADTOT = NSUB * ECHUNKS * CH
ROWS_PER_SUB = NPAD // NSUB
BN = 1000
NBLK = N // BN

f32 = jnp.float32
i32 = jnp.int32

_SC_MESH = dict(core_axis_name="c", subcore_axis_name="s",
                num_cores=NCORE, num_subcores=NSUB)


def _zero_rows(ref, nrows, width):
    zero16 = jnp.zeros((16,), f32)

    def zrow(r, _):
        for k in range(width // 16):
            ref[r, pl.ds(k * 16, 16)] = zero16
        return 0

    lax.fori_loop(0, nrows, zrow, 0)


def _sc_init_body(comb0, comb1, xi_hbm, h0_out, h1_out,
                  xi_v, rows_v, sem):
    c = lax.axis_index("c")
    s = lax.axis_index("s")
    pltpu.sync_copy(xi_hbm.at[s], xi_v)

    def gather_phase(tbl, hout):
        for j in range(NCHUNK_N):
            pltpu.sync_copy(tbl.at[xi_v.at[j]], rows_v)
            pltpu.sync_copy(rows_v, hout.at[pl.ds((s * NCHUNK_N + j) * CH, CH)])

    pl.when(c == 0)(lambda: gather_phase(comb0, h0_out))
    pl.when(c == 1)(lambda: gather_phase(comb1, h1_out))


def _sc_init(comb0, comb1, xi):
    kfn = pl.kernel(
        _sc_init_body,
        out_type=(
            jax.ShapeDtypeStruct((NPAD, HALF), f32),
            jax.ShapeDtypeStruct((NPAD, HALF), f32),
        ),
        mesh=plsc.VectorSubcoreMesh(**_SC_MESH),
        scratch_types=(
            pltpu.VMEM((NCHUNK_N, CH), i32),
            pltpu.VMEM((CH, HALF), f32),
            pltpu.SemaphoreType.DMA,
        ),
    )
    return kfn(comb0, comb1, xi)


def _sc_layer_body(hx0, hx1, packed_hbm, s0_out, s1_out,
                   packed_r, gidx_r, dst_r, rows_v, s_sh, sem):
    c = lax.axis_index("c")
    s = lax.axis_index("s")
    _zero_rows(rows_v, CH, HALF)
    for k in range(NCHUNK_N):
        pltpu.sync_copy(rows_v, s_sh.at[pl.ds(s * ROWS_PER_SUB + k * CH, CH)])
    plsc.subcore_barrier()

    def run(hx_ref):
        def step(j, _):
            pltpu.sync_copy(packed_hbm.at[s, j], packed_r)
            for k in range(CH // 16):
                sl = pl.ds(k * 16, 16)
                v = packed_r[sl]
                gidx_r[sl] = lax.shift_right_logical(v, 14)
                dst_r[sl] = jnp.bitwise_and(v, 0x3FFF)
            pltpu.sync_copy(hx_ref.at[gidx_r], rows_v)
            pltpu.sync_copy(rows_v, s_sh.at[dst_r], add=True)
            return 0

        lax.fori_loop(0, ECHUNKS, step, 0)

    pl.when(c == 0)(lambda: run(hx0))
    pl.when(c == 1)(lambda: run(hx1))
    plsc.subcore_barrier()

    def out(s_ref):
        pltpu.sync_copy(s_sh.at[pl.ds(s * ROWS_PER_SUB, ROWS_PER_SUB)],
                        s_ref.at[pl.ds(s * ROWS_PER_SUB, ROWS_PER_SUB)])

    pl.when(c == 0)(lambda: out(s0_out))
    pl.when(c == 1)(lambda: out(s1_out))


def _sc_layer(hx0, hx1, packed):
    kfn = pl.kernel(
        _sc_layer_body,
        out_type=(
            jax.ShapeDtypeStruct((NPAD, HALF), f32),
            jax.ShapeDtypeStruct((NPAD, HALF), f32),
        ),
        mesh=plsc.VectorSubcoreMesh(**_SC_MESH),
        scratch_types=(
            pltpu.VMEM((CH,), i32),
            pltpu.VMEM((CH,), i32),
            pltpu.VMEM((CH,), i32),
            pltpu.VMEM((CH, HALF), f32),
            pltpu.VMEM_SHARED((NPAD, HALF), f32),
            pltpu.SemaphoreType.DMA,
        ),
    )
    return kfn(hx0, hx1, packed)


def _k1_body(s0, s1, w1, b1, w2, b2, z_out, sums, comp):
    i = pl.program_id(0)
    agg = jnp.concatenate([s0[...], s1[...]], axis=1)
    z1 = jnp.maximum(jnp.dot(agg, w1[...], preferred_element_type=f32) + b1[...], 0.0)
    z2 = jnp.dot(z1, w2[...], preferred_element_type=f32) + b2[...]
    z_out[...] = z2

    @pl.when(i == 0)
    def _():
        sums[...] = jnp.zeros_like(sums)
        comp[...] = jnp.zeros_like(comp)

    bs = jnp.sum(z2, axis=0, keepdims=True)
    y = bs - comp[...]
    t = sums[...] + y
    comp[...] = (t - sums[...]) - y
    sums[...] = t


def _tc_mlp(s0, s1, w1, b1, w2, b2):
    blk = lambda r, cdim: pl.BlockSpec((r, cdim), lambda i: (i, 0))
    full = lambda a, b: pl.BlockSpec((a, b), lambda i: (0, 0))
    return pl.pallas_call(
        _k1_body,
        grid=(NBLK,),
        in_specs=[
            blk(BN, HALF), blk(BN, HALF),
            full(EMB, FEAT), full(1, FEAT),
            full(FEAT, EMB), full(1, EMB),
        ],
        out_specs=[
            pl.BlockSpec((BN, EMB), lambda i: (i, 0)),
            full(1, EMB),
        ],
        out_shape=[
            jax.ShapeDtypeStruct((N, EMB), f32),
            jax.ShapeDtypeStruct((1, EMB), f32),
        ],
        scratch_shapes=[pltpu.VMEM((1, EMB), f32)],
    )(s0, s1, w1, b1, w2, b2)


NXP = NPAD * 16


def _expand_body(h0, h1, ct0, ct1, hx0_out, hx1_out):
    hx0_out[...] = (h0[...][:, None, :] + ct0[...][None, :, :]).reshape(BN * 16, HALF)
    hx1_out[...] = (h1[...][:, None, :] + ct1[...][None, :, :]).reshape(BN * 16, HALF)


def _tc_expand(h0, h1, ct0, ct1):
    full = lambda a, b: pl.BlockSpec((a, b), lambda i: (0, 0))
    return pl.pallas_call(
        _expand_body,
        grid=(NBLK,),
        in_specs=[
            pl.BlockSpec((BN, HALF), lambda i: (i, 0)),
            pl.BlockSpec((BN, HALF), lambda i: (i, 0)),
            full(16, HALF), full(16, HALF),
        ],
        out_specs=[
            pl.BlockSpec((BN * 16, HALF), lambda i: (i, 0)),
            pl.BlockSpec((BN * 16, HALF), lambda i: (i, 0)),
        ],
        out_shape=[
            jax.ShapeDtypeStruct((NXP, HALF), f32),
            jax.ShapeDtypeStruct((NXP, HALF), f32),
        ],
    )(h0, h1, ct0, ct1)


def _k2_body(z, sums, gamma, beta, h0_out, h1_out, ssq, scomp, *, relu):
    i = pl.program_id(0)
    mean = sums[...] / N

    @pl.when(i == 0)
    def _():
        ssq[...] = jnp.zeros_like(ssq)
        scomp[...] = jnp.zeros_like(scomp)

    @pl.when(i < NBLK)
    def _():
        d = z[...] - mean
        bs = jnp.sum(d * d, axis=0, keepdims=True)
        y = bs - scomp[...]
        t = ssq[...] + y
        scomp[...] = (t - ssq[...]) - y
        ssq[...] = t

    @pl.when(i >= NBLK)
    def _():
        var = ssq[...] / N
        hn = (z[...] - mean) / jnp.sqrt(var + EPS) * gamma[...] + beta[...]
        if relu:
            hn = jnp.maximum(hn, 0.0)
        h0_out[...] = hn[:, :HALF]
        h1_out[...] = hn[:, HALF:]


def _k2x_body(z, sums, gamma, beta, ct0, ct1, hx0_out, hx1_out, ssq, scomp):
    i = pl.program_id(0)
    mean = sums[...] / N

    @pl.when(i == 0)
    def _():
        ssq[...] = jnp.zeros_like(ssq)
        scomp[...] = jnp.zeros_like(scomp)

    @pl.when(i < NBLK)
    def _():
        d = z[...] - mean
        bs = jnp.sum(d * d, axis=0, keepdims=True)
        y = bs - scomp[...]
        t = ssq[...] + y
        scomp[...] = (t - ssq[...]) - y
        ssq[...] = t

    @pl.when(i >= NBLK)
    def _():
        var = ssq[...] / N
        hn = (z[...] - mean) / jnp.sqrt(var + EPS) * gamma[...] + beta[...]
        hn = jnp.maximum(hn, 0.0)
        hx0_out[...] = (hn[:, :HALF][:, None, :] +
                        ct0[...][None, :, :]).reshape(BN * 16, HALF)
        hx1_out[...] = (hn[:, HALF:][:, None, :] +
                        ct1[...][None, :, :]).reshape(BN * 16, HALF)


def _tc_bn_expand(z, sums, gamma, beta, ct0, ct1):
    full = lambda a, b: pl.BlockSpec((a, b), lambda i: (0, 0))
    return pl.pallas_call(
        _k2x_body,
        grid=(2 * NBLK,),
        in_specs=[
            pl.BlockSpec((BN, EMB), lambda i: (i % NBLK, 0)),
            full(1, EMB), full(1, EMB), full(1, EMB),
            full(16, HALF), full(16, HALF),
        ],
        out_specs=[
            pl.BlockSpec((BN * 16, HALF), lambda i: (i % NBLK, 0)),
            pl.BlockSpec((BN * 16, HALF), lambda i: (i % NBLK, 0)),
        ],
        out_shape=[
            jax.ShapeDtypeStruct((NXP, HALF), f32),
            jax.ShapeDtypeStruct((NXP, HALF), f32),
        ],
        scratch_shapes=[pltpu.VMEM((1, EMB), f32), pltpu.VMEM((1, EMB), f32)],
    )(z, sums, gamma, beta, ct0, ct1)


def _tc_bn(z, sums, gamma, beta, relu):
    full = lambda a, b: pl.BlockSpec((a, b), lambda i: (0, 0))
    return pl.pallas_call(
        functools.partial(_k2_body, relu=relu),
        grid=(2 * NBLK,),
        in_specs=[
            pl.BlockSpec((BN, EMB), lambda i: (i % NBLK, 0)),
            full(1, EMB), full(1, EMB), full(1, EMB),
        ],
        out_specs=[
            pl.BlockSpec((BN, HALF), lambda i: (i % NBLK, 0)),
            pl.BlockSpec((BN, HALF), lambda i: (i % NBLK, 0)),
        ],
        out_shape=[
            jax.ShapeDtypeStruct((NPAD, HALF), f32),
            jax.ShapeDtypeStruct((NPAD, HALF), f32),
        ],
        scratch_shapes=[pltpu.VMEM((1, EMB), f32), pltpu.VMEM((1, EMB), f32)],
    )(z, sums, gamma, beta)


def _softplus(x):
    return jnp.maximum(x, 0.0) + jnp.log1p(jnp.exp(-jnp.abs(x)))


def _kf_body(h0, h1, bat, fw, fb, w1, b1, w2, b2, w3p, b3p,
             hg_out, pred_out, acc0, acc1, cnt):
    i = pl.program_id(0)
    bcol = bat[0]
    iota = lax.broadcasted_iota(i32, (BN, NUM_GRAPHS), 1)
    ohf = (bcol == iota).astype(f32)

    @pl.when(i == 0)
    def _():
        acc0[...] = jnp.zeros_like(acc0)
        acc1[...] = jnp.zeros_like(acc1)
        cnt[...] = jnp.zeros_like(cnt)

    dn = (((0,), (0,)), ((), ()))
    hp = lax.Precision.HIGHEST
    acc0[...] += lax.dot_general(ohf, h0[...], dn, precision=hp,
                                 preferred_element_type=f32)
    acc1[...] += lax.dot_general(ohf, h1[...], dn, precision=hp,
                                 preferred_element_type=f32)
    cnt[...] += lax.dot_general(ohf, jnp.ones((BN, HALF), f32), dn, precision=hp,
                                preferred_element_type=f32)

    @pl.when(i == NBLK - 1)
    def _():
        cmax = jnp.maximum(cnt[...][:, 0:1], 1.0)
        pool = jnp.concatenate([acc0[...], acc1[...]], axis=1) / cmax
        hg = jnp.dot(pool, fw[...], preferred_element_type=f32) + fb[...]
        hg_out[...] = hg
        p = _softplus(jnp.dot(hg, w1[...], preferred_element_type=f32) + b1[...])
        p = _softplus(jnp.dot(p, w2[...], preferred_element_type=f32) + b2[...])
        pred_out[...] = jnp.dot(p, w3p[...], preferred_element_type=f32) + b3p[...]


def _tc_pool_head(h0, h1, bat3, fw, fb, w1, b1, w2, b2, w3p, b3p):
    full = lambda a, b: pl.BlockSpec((a, b), lambda i: (0, 0))
    return pl.pallas_call(
        _kf_body,
        grid=(NBLK,),
        in_specs=[
            pl.BlockSpec((BN, HALF), lambda i: (i, 0)),
            pl.BlockSpec((BN, HALF), lambda i: (i, 0)),
            pl.BlockSpec((1, BN, 1), lambda i: (i, 0, 0)),
            full(EMB, FEAT), full(1, FEAT),
            full(FEAT, FEAT // 2), full(1, FEAT // 2),
            full(FEAT // 2, FEAT // 2), full(1, FEAT // 2),
            full(FEAT // 2, HALF), full(1, HALF),
        ],
        out_specs=[
            full(NUM_GRAPHS, FEAT),
            full(NUM_GRAPHS, HALF),
        ],
        out_shape=[
            jax.ShapeDtypeStruct((NUM_GRAPHS, FEAT), f32),
            jax.ShapeDtypeStruct((NUM_GRAPHS, HALF), f32),
        ],
        scratch_shapes=[
            pltpu.VMEM((NUM_GRAPHS, HALF), f32),
            pltpu.VMEM((NUM_GRAPHS, HALF), f32),
            pltpu.VMEM((NUM_GRAPHS, HALF), f32),
        ],
    )(h0, h1, bat3, fw, fb, w1, b1, w2, b2, w3p, b3p)


def kernel(x, edge_index, edge_attr, batch, params):
    comb = (params['x_emb1'][:, None, :] + params['x_emb2'][None, :, :])
    comb = comb.reshape(-1, EMB)
    comb0 = comb[:, :HALF]
    comb1 = comb[:, HALF:]
    nct = params['x_emb2'].shape[0]
    xi = (x[:, 0] * nct + x[:, 1]).astype(i32)
    xi = jnp.concatenate([xi, jnp.zeros((NPAD - N,), i32)])
    xi = xi.reshape(NSUB, NCHUNK_N, CH)

    self_idx = jnp.arange(N, dtype=i32)
    full_src = jnp.concatenate([edge_index[0].astype(i32), self_idx])
    full_dst = jnp.concatenate([edge_index[1].astype(i32), self_idx])
    full_eidx = jnp.concatenate(
        [(edge_attr[:, 0] * 3 + edge_attr[:, 1]).astype(i32),
         jnp.full((N,), 12, i32)])
    order = jnp.argsort(full_dst, stable=True)
    pad_e = EPADTOT - (E + N)
    pad_word = (((NPAD - 1) * 16 + 15) << 14) | (NPAD - 1)
    packed = (((full_src[order] * 16 + full_eidx[order]) << 14) | full_dst[order])
    packed = jnp.concatenate([packed,
                              jnp.full((pad_e,), pad_word, i32)]).reshape(NSUB, ECHUNKS, CH)

    bat3 = batch.astype(i32).reshape(NBLK, BN, 1)

    def ctabs(l):
        e1 = params['edge_emb1'][l]
        e2 = params['edge_emb2'][l]
        ctab = (e1[:, None, :] + e2[None, :, :]).reshape(-1, EMB)
        ctab = jnp.concatenate([ctab, jnp.zeros((1, EMB), f32)])
        return ctab[:, :HALF], ctab[:, HALF:]

    h0, h1 = _sc_init(comb0, comb1, xi)
    hx0, hx1 = _tc_expand(h0, h1, *ctabs(0))

    for l in range(NUM_LAYER):
        w1 = params['w1'][l]
        b1 = params['b1'][l].reshape(1, -1)
        w2 = params['w2'][l]
        b2 = params['b2'][l].reshape(1, -1)
        gamma = params['bn_gamma'][l].reshape(1, -1)
        beta = params['bn_beta'][l].reshape(1, -1)

        s0, s1 = _sc_layer(hx0, hx1, packed)
        z, sums = _tc_mlp(s0, s1, w1, b1, w2, b2)
        if l != NUM_LAYER - 1:
            hx0, hx1 = _tc_bn_expand(z, sums, gamma, beta, *ctabs(l + 1))
        else:
            h0, h1 = _tc_bn(z, sums, gamma, beta, relu=False)

    fw = params['feat_w']
    w3p = jnp.pad(params['head_w3'], ((0, 0), (0, HALF - 1)))
    b3p = jnp.pad(params['head_b3'].reshape(1, 1), ((0, 0), (0, HALF - 1)))
    hg, pred_full = _tc_pool_head(
        h0, h1, bat3,
        fw, params['feat_b'].reshape(1, -1),
        params['head_w1'], params['head_b1'].reshape(1, -1),
        params['head_w2'], params['head_b2'].reshape(1, -1),
        w3p, b3p)
    return (hg, pred_full[:, :1])
